# baseline (speedup 1.0000x reference)
import functools

import jax
import jax.numpy as jnp
from jax import lax
from jax.experimental import pallas as pl
from jax.experimental.pallas import tpu as pltpu
from jax.experimental.pallas import tpu_sc as plsc

_SUB = 64
_NCHUNK = 6


@functools.cache
def _make_gather(num_rows: int, lanes: int):
    info = plsc.get_sparse_core_info()
    nw = info.num_cores * info.num_subcores
    nc = info.num_cores
    mesh = plsc.VectorSubcoreMesh(core_axis_name="c", subcore_axis_name="s")

    @functools.partial(
        pl.kernel,
        mesh=mesh,
        out_type=jax.ShapeDtypeStruct((nw, _NCHUNK, _SUB, lanes), jnp.float32),
        scratch_types=[
            pltpu.VMEM((_NCHUNK, _SUB), jnp.int32),
            pltpu.VMEM((_NCHUNK, _SUB, lanes), jnp.float32),
            pltpu.SemaphoreType.DMA,
            pltpu.SemaphoreType.DMA,
        ],
    )
    def gather_kernel(table_hbm, idx_hbm, out_hbm, idx_v, rows_v, sem_g, sem_s):
        wid = lax.axis_index("s") * nc + lax.axis_index("c")
        pltpu.sync_copy(idx_hbm.at[wid], idx_v)
        gathers = [None] * _NCHUNK
        stores = []
        for q in range(2):
            gathers[q] = pltpu.async_copy(
                table_hbm.at[idx_v.at[q]], rows_v.at[q], sem_g)
        for q in range(_NCHUNK):
            gathers[q].wait()
            if q + 2 < _NCHUNK:
                gathers[q + 2] = pltpu.async_copy(
                    table_hbm.at[idx_v.at[q + 2]], rows_v.at[q + 2], sem_g)
            stores.append(
                pltpu.async_copy(rows_v.at[q], out_hbm.at[wid].at[q], sem_s))
        for st in stores:
            st.wait()

    return gather_kernel


def kernel(class_number, embedding):
    num_classes, pts, ch = embedding.shape
    batch = class_number.shape[0]
    table = jnp.moveaxis(embedding, 2, 0).reshape(num_classes * ch, pts)
    idx = class_number.astype(jnp.int32)
    idx3 = idx[None, :] + (jnp.arange(ch, dtype=jnp.int32) * num_classes)[:, None]
    nw = 32
    assert ch * batch == nw * _NCHUNK * _SUB
    idx_m = idx3.reshape(nw, _NCHUNK, _SUB)
    out = _make_gather(num_classes * ch, pts)(table, idx_m)
    return jnp.moveaxis(out.reshape(ch, batch, pts), 0, 2)


# speedup vs baseline: 1.0340x; 1.0340x over previous
import functools

import jax
import jax.numpy as jnp
from jax import lax
from jax.experimental import pallas as pl
from jax.experimental.pallas import tpu as pltpu
from jax.experimental.pallas import tpu_sc as plsc

_SUB = 128
_NCHUNK = 3


@functools.cache
def _make_gather(num_rows: int, lanes: int):
    info = plsc.get_sparse_core_info()
    nw = info.num_cores * info.num_subcores
    nc = info.num_cores
    mesh = plsc.VectorSubcoreMesh(core_axis_name="c", subcore_axis_name="s")

    @functools.partial(
        pl.kernel,
        mesh=mesh,
        out_type=jax.ShapeDtypeStruct((nw, _NCHUNK, _SUB, lanes), jnp.float32),
        scratch_types=[
            pltpu.VMEM((_NCHUNK, _SUB), jnp.int32),
            pltpu.VMEM((_NCHUNK, _SUB, lanes), jnp.float32),
            pltpu.SemaphoreType.DMA,
            pltpu.SemaphoreType.DMA,
        ],
    )
    def gather_kernel(table_hbm, idx_hbm, out_hbm, idx_v, rows_v, sem_g, sem_s):
        wid = lax.axis_index("s") * nc + lax.axis_index("c")
        pltpu.sync_copy(idx_hbm.at[wid], idx_v)
        gathers = [
            pltpu.async_copy(table_hbm.at[idx_v.at[q]], rows_v.at[q], sem_g)
            for q in range(_NCHUNK)
        ]
        stores = []
        for q in range(_NCHUNK):
            gathers[q].wait()
            stores.append(
                pltpu.async_copy(rows_v.at[q], out_hbm.at[wid].at[q], sem_s))
        for st in stores:
            st.wait()

    return gather_kernel


def kernel(class_number, embedding):
    num_classes, pts, ch = embedding.shape
    batch = class_number.shape[0]
    table = jnp.moveaxis(embedding, 2, 0).reshape(num_classes * ch, pts)
    idx = class_number.astype(jnp.int32)
    idx3 = idx[None, :] + (jnp.arange(ch, dtype=jnp.int32) * num_classes)[:, None]
    nw = 32
    assert ch * batch == nw * _NCHUNK * _SUB
    idx_m = idx3.reshape(nw, _NCHUNK, _SUB)
    out = _make_gather(num_classes * ch, pts)(table, idx_m)
    return jnp.moveaxis(out.reshape(ch, batch, pts), 0, 2)


# speedup vs baseline: 1.0520x; 1.0174x over previous
import functools

import jax
import jax.numpy as jnp
from jax import lax
from jax.experimental import pallas as pl
from jax.experimental.pallas import tpu as pltpu
from jax.experimental.pallas import tpu_sc as plsc

_SUB = 128
_NCHUNK = 3


@functools.cache
def _make_gather(num_rows: int, lanes: int):
    info = plsc.get_sparse_core_info()
    nw = info.num_cores * info.num_subcores
    nc = info.num_cores
    mesh = plsc.VectorSubcoreMesh(core_axis_name="c", subcore_axis_name="s")

    @functools.partial(
        pl.kernel,
        mesh=mesh,
        out_type=jax.ShapeDtypeStruct((nw, _NCHUNK, _SUB, lanes), jnp.float32),
        scratch_types=[
            pltpu.VMEM((_NCHUNK, _SUB), jnp.int32),
            pltpu.VMEM((_NCHUNK, _SUB, lanes), jnp.float32),
            pltpu.SemaphoreType.DMA,
        ],
    )
    def gather_kernel(table_hbm, idx_hbm, out_hbm, idx_v, rows_v, sem_g):
        wid = lax.axis_index("s") * nc + lax.axis_index("c")
        pltpu.sync_copy(idx_hbm.at[wid], idx_v)
        gathers = [
            pltpu.async_copy(table_hbm.at[idx_v.at[q]], rows_v.at[q], sem_g)
            for q in range(_NCHUNK)
        ]
        for g in gathers:
            g.wait()
        pltpu.sync_copy(rows_v, out_hbm.at[wid])

    return gather_kernel


def kernel(class_number, embedding):
    num_classes, pts, ch = embedding.shape
    batch = class_number.shape[0]
    table = jnp.moveaxis(embedding, 2, 0).reshape(num_classes * ch, pts)
    idx = class_number.astype(jnp.int32)
    idx3 = idx[None, :] + (jnp.arange(ch, dtype=jnp.int32) * num_classes)[:, None]
    nw = 32
    assert ch * batch == nw * _NCHUNK * _SUB
    idx_m = idx3.reshape(nw, _NCHUNK, _SUB)
    out = _make_gather(num_classes * ch, pts)(table, idx_m)
    return jnp.moveaxis(out.reshape(ch, batch, pts), 0, 2)
